# Initial kernel scaffold; baseline (speedup 1.0000x reference)
import jax
import jax.numpy as jnp
from jax.experimental import pallas as pl


def kernel(x, W1, b1, W2, b2, W3, b3, proxies, Wd, bd, eps):
    raise NotImplementedError("write your pallas kernel here")


# speedup vs baseline: 5.6310x; 5.6310x over previous
import functools

import jax
import jax.numpy as jnp
from jax import lax
from jax.experimental import pallas as pl
from jax.experimental.pallas import tpu as pltpu
from jax.experimental.pallas import tpu_sc as plsc

B, N, XD, ZD, NC, SN, TOPK = 64, 8192, 64, 64, 4, 50, 256
P = 2 * NC
F = N * P
TN = 2048
CAND = 512
NCORES, NSUB = 2, 16
NW = NCORES * NSUB
BISECT_ITERS = 30

_f32 = jnp.float32
_i32 = jnp.int32


def _proxy_kernel(proxies_ref, eps_ref, wd_ref, bd_ref,
                  mu_ref, sigma_ref, zpn_ref, dlog_ref):
    pr = proxies_ref[...]
    mu = pr[:, :ZD]
    x = pr[:, ZD:]
    sigma = jnp.maximum(x, 0.0) + jnp.log1p(jnp.exp(-jnp.abs(x)))
    mu_ref[...] = mu
    sigma_ref[...] = sigma
    eps = eps_ref[...]
    zps = mu[:, None, :] + sigma[:, None, :] * eps
    zp = jnp.mean(zps, axis=1)
    nrm = jnp.sqrt(jnp.sum(zp * zp, axis=1, keepdims=True))
    zpn_ref[...] = zp / jnp.maximum(nrm, 1e-12)
    flat = zps.reshape(P * SN, ZD)
    logits = jnp.dot(flat, wd_ref[...], preferred_element_type=_f32)
    logits = logits + bd_ref[...]
    dlog_ref[...] = jnp.mean(logits.reshape(P, SN, NC), axis=1)


def _run_proxy(proxies, eps, Wd, bd):
    return pl.pallas_call(
        _proxy_kernel,
        out_shape=(
            jax.ShapeDtypeStruct((P, ZD), _f32),
            jax.ShapeDtypeStruct((P, ZD), _f32),
            jax.ShapeDtypeStruct((P, ZD), _f32),
            jax.ShapeDtypeStruct((P, NC), _f32),
        ),
    )(proxies, eps, Wd, bd.reshape(1, NC))


def _mlp_kernel(x_ref, w1_ref, b1_ref, w2_ref, b2_ref, w3_ref, b3_ref,
                zpn_ref, z_ref, att_ref):
    xt = x_ref[0]
    h = jnp.maximum(jnp.dot(xt, w1_ref[...], preferred_element_type=_f32)
                    + b1_ref[...], 0.0)
    h = jnp.maximum(jnp.dot(h, w2_ref[...], preferred_element_type=_f32)
                    + b2_ref[...], 0.0)
    z = jnp.dot(h, w3_ref[...], preferred_element_type=_f32) + b3_ref[...]
    z_ref[0] = jnp.concatenate([z, z], axis=1)
    nrm = jnp.sqrt(jnp.sum(z * z, axis=1, keepdims=True))
    zn = z / jnp.maximum(nrm, 1e-12)
    att = lax.dot_general(zn, zpn_ref[...], (((1,), (1,)), ((), ())),
                          preferred_element_type=_f32)
    att_ref[0] = att.T


def _run_mlp(x, W1, b1, W2, b2, W3, b3, zpn):
    grid = (B, N // TN)
    return pl.pallas_call(
        _mlp_kernel,
        grid=grid,
        in_specs=[
            pl.BlockSpec((1, TN, XD), lambda b, j: (b, j, 0)),
            pl.BlockSpec((XD, 2 * ZD), lambda b, j: (0, 0)),
            pl.BlockSpec((1, 2 * ZD), lambda b, j: (0, 0)),
            pl.BlockSpec((2 * ZD, 2 * ZD), lambda b, j: (0, 0)),
            pl.BlockSpec((1, 2 * ZD), lambda b, j: (0, 0)),
            pl.BlockSpec((2 * ZD, ZD), lambda b, j: (0, 0)),
            pl.BlockSpec((1, ZD), lambda b, j: (0, 0)),
            pl.BlockSpec((P, ZD), lambda b, j: (0, 0)),
        ],
        out_specs=[
            pl.BlockSpec((1, TN, 2 * ZD), lambda b, j: (b, j, 0)),
            pl.BlockSpec((1, P, TN), lambda b, j: (b, 0, j)),
        ],
        out_shape=(
            jax.ShapeDtypeStruct((B, N, 2 * ZD), _f32),
            jax.ShapeDtypeStruct((B, P, N), _f32),
        ),
        compiler_params=pltpu.CompilerParams(
            dimension_semantics=("parallel", "parallel")),
    )(x, W1, b1.reshape(1, -1), W2, b2.reshape(1, -1), W3, b3.reshape(1, -1),
      zpn)


def _bisect_kernel(att_ref, thr_ref):
    a = att_ref[...]
    rb = a.shape[0]
    k = jnp.float32(TOPK)

    def body(_, carry):
        lo, hi = carry
        mid = 0.5 * (lo + hi)
        cnt = jnp.sum((a > mid).astype(_f32), axis=1, keepdims=True)
        take = cnt >= k
        return jnp.where(take, mid, lo), jnp.where(take, hi, mid)

    lo0 = jnp.full((rb, 1), -1.01, _f32)
    hi0 = jnp.full((rb, 1), 1.01, _f32)
    lo, _ = lax.fori_loop(0, BISECT_ITERS, body, (lo0, hi0))
    thr_ref[...] = jnp.broadcast_to(lo, (rb, 16))


def _run_bisect(att_flat):
    rb = 8
    return pl.pallas_call(
        _bisect_kernel,
        grid=(B // rb,),
        in_specs=[pl.BlockSpec((rb, F), lambda i: (i, 0))],
        out_specs=pl.BlockSpec((rb, 16), lambda i: (i, 0)),
        out_shape=jax.ShapeDtypeStruct((B, 16), _f32),
        compiler_params=pltpu.CompilerParams(
            dimension_semantics=("arbitrary",)),
    )(att_flat)


def _sc_compact_body(att_hbm, thr_hbm, cv_hbm, ci_hbm,
                     buf_v, thr_v, cv_v, ci_v):
    wid = lax.axis_index("s") * NCORES + lax.axis_index("c")
    half = F // 2
    lane = lax.iota(_i32, 16)

    def one_batch(bi, _):
        b = wid * (B // NW) + bi
        pltpu.sync_copy(thr_hbm.at[b], thr_v)
        thrv = thr_v[...]

        def init_body(i, _c):
            cv_v[pl.ds(i * 16, 16)] = jnp.full((16,), -2.0, _f32)
            ci_v[pl.ds(i * 16, 16)] = jnp.zeros((16,), _i32)
            return 0

        lax.fori_loop(0, CAND // 16, init_body, 0)

        def scan_half(h, off):
            pltpu.sync_copy(att_hbm.at[b, pl.ds(h * half, half)], buf_v)

            def chunk(i, off):
                v = buf_v[pl.ds(i * 16, 16)]
                m = v > thrv

                def append(off):
                    mi = m.astype(_i32)
                    pos = jnp.minimum(off + plsc.cumsum(mi) - 1, CAND - 1)
                    fidx = h * half + i * 16 + lane
                    plsc.store_scatter(cv_v, [pos], v, mask=m)
                    plsc.store_scatter(ci_v, [pos], fidx, mask=m)
                    return off + jnp.sum(mi)

                return lax.cond(jnp.any(m), append, lambda off: off, off)

            return lax.fori_loop(0, half // 16, chunk, off)

        lax.fori_loop(0, 2, scan_half, 0)
        pltpu.sync_copy(cv_v, cv_hbm.at[b])
        pltpu.sync_copy(ci_v, ci_hbm.at[b])
        return 0

    lax.fori_loop(0, B // NW, one_batch, 0)


def _run_sc_compact(att_flat, thr):
    mesh = plsc.VectorSubcoreMesh(core_axis_name="c", subcore_axis_name="s",
                                  num_cores=NCORES, num_subcores=NSUB)
    return pl.kernel(
        _sc_compact_body,
        out_type=(
            jax.ShapeDtypeStruct((B, CAND), _f32),
            jax.ShapeDtypeStruct((B, CAND), _i32),
        ),
        mesh=mesh,
        scratch_types=[
            pltpu.VMEM((F // 2,), _f32),
            pltpu.VMEM((16,), _f32),
            pltpu.VMEM((CAND,), _f32),
            pltpu.VMEM((CAND,), _i32),
        ],
        compiler_params=pltpu.CompilerParams(needs_layout_passes=False),
    )(att_flat, thr)


def _rotl(x, s):
    return jnp.concatenate([x[:, s:], x[:, :s]], axis=1)


def _bitonic_desc_ref(vrf, frf):
    L = CAND
    ii = lax.broadcasted_iota(_i32, (1, L), 1)
    k = 2
    while k <= L:
        j = k // 2
        while j >= 1:
            v = vrf[...]
            f = frf[...]
            low = (ii & j) == 0
            fwd = (ii & k) == 0
            pv = jnp.where(low, _rotl(v, j), _rotl(v, L - j))
            pf = jnp.where(low, _rotl(f, j), _rotl(f, L - j))
            i_first = (v > pv) | ((v == pv) & (f < pf))
            take_mine = jnp.logical_xor(jnp.logical_xor(low, i_first), fwd)
            vrf[...] = jnp.where(take_mine, v, pv)
            frf[...] = jnp.where(take_mine, f, pf)
            j //= 2
        k *= 2


def _select_kernel(cv_ref, ci_ref, mu_ref, sg_ref,
                   gidx_ref, mut_ref, sgt_ref, vs_ref, fs_ref):
    v = cv_ref[...]
    fi = ci_ref[...]
    vs_ref[...] = v
    fs_ref[...] = fi
    _bitonic_desc_ref(vs_ref, fs_ref)
    f2 = fs_ref[...]
    n_sel = f2[:, :TOPK] & (N - 1)
    boff = lax.broadcasted_iota(_i32, (B, TOPK), 0) * N
    gidx_ref[...] = n_sel + boff
    vs_ref[...] = v
    fs_ref[...] = (fi & (N - 1)) * P + (fi >> 13)
    _bitonic_desc_ref(vs_ref, fs_ref)
    p_sel = fs_ref[...][:, :TOPK] & (P - 1)
    counts = [jnp.sum((p_sel == p).astype(_f32), axis=1, keepdims=True)
              for p in range(P)]
    best = counts[0]
    besti = jnp.zeros((B, 1), _i32)
    for p in range(1, P):
        m = counts[p] > best
        best = jnp.where(m, counts[p], best)
        besti = jnp.where(m, p, besti)
    mu = mu_ref[...]
    sg = sg_ref[...]
    mut = jnp.zeros((B, ZD), _f32)
    sgt = jnp.zeros((B, ZD), _f32)
    for p in range(P):
        m = besti == p
        mut = mut + jnp.where(m, mu[p:p + 1, :], 0.0)
        sgt = sgt + jnp.where(m, sg[p:p + 1, :], 0.0)
    mut_ref[...] = mut
    sgt_ref[...] = sgt


def _run_select(cv, ci, mu, sigma):
    return pl.pallas_call(
        _select_kernel,
        out_shape=(
            jax.ShapeDtypeStruct((B, TOPK), _i32),
            jax.ShapeDtypeStruct((B, ZD), _f32),
            jax.ShapeDtypeStruct((B, ZD), _f32),
        ),
        scratch_shapes=[
            pltpu.VMEM((B, CAND), _f32),
            pltpu.VMEM((B, CAND), _i32),
        ],
    )(cv, ci, mu, sigma)


def _sc_gather_body(z_hbm, idx_hbm, out_hbm, idx_v, rows_v, sem):
    wid = lax.axis_index("s") * NCORES + lax.axis_index("c")
    per = (B * TOPK) // NW
    base = wid * per
    pltpu.sync_copy(idx_hbm.at[pl.ds(base, per)], idx_v)
    pltpu.async_copy(z_hbm.at[idx_v], rows_v, sem).wait()
    pltpu.sync_copy(rows_v, out_hbm.at[pl.ds(base, per)])


def _run_sc_gather(z_rows, gidx_flat):
    per = (B * TOPK) // NW
    mesh = plsc.VectorSubcoreMesh(core_axis_name="c", subcore_axis_name="s",
                                  num_cores=NCORES, num_subcores=NSUB)
    return pl.kernel(
        _sc_gather_body,
        out_type=jax.ShapeDtypeStruct((B * TOPK, 2 * ZD), _f32),
        mesh=mesh,
        scratch_types=[
            pltpu.VMEM((per,), _i32),
            pltpu.VMEM((per, 2 * ZD), _f32),
            pltpu.SemaphoreType.DMA,
        ],
        compiler_params=pltpu.CompilerParams(needs_layout_passes=False),
    )(z_rows, gidx_flat)


_TMP_JAX_COMPACT = False
_TMP_JAX_GATHER = False


def _jax_compact(att_flat, thr):
    val, idx = lax.top_k(att_flat, CAND)
    keep = val > thr[:, :1]
    cv = jnp.where(keep, val, -2.0)
    ci = jnp.where(keep, idx, 0)
    order = jnp.argsort(ci, axis=1)
    return (jnp.take_along_axis(cv, order, axis=1),
            jnp.take_along_axis(ci, order, axis=1))


def kernel(x, W1, b1, W2, b2, W3, b3, proxies, Wd, bd, eps):
    mu, sigma, zpn_unused, dlog = _run_proxy(proxies, eps, Wd, bd)
    mu_p = proxies[:, :ZD]
    sg_p = jax.nn.softplus(proxies[:, ZD:])
    zps = mu_p[:, None, :] + sg_p[:, None, :] * eps
    zp = jnp.mean(zps, axis=1)
    zpn = zp / jnp.maximum(jnp.linalg.norm(zp, axis=1, keepdims=True), 1e-12)
    z, att_t = _run_mlp(x, W1, b1, W2, b2, W3, b3, zpn)
    att_flat = att_t.reshape(B, F)
    thr = _run_bisect(att_flat)
    if _TMP_JAX_COMPACT:
        cv, ci = _jax_compact(att_flat, thr)
    else:
        cv, ci = _run_sc_compact(att_flat, thr)
    gidx, mu_topk, sigma_topk = _run_select(cv, ci, mu, sigma)
    if _TMP_JAX_GATHER:
        wide = jnp.take(z.reshape(B * N, 2 * ZD), gidx.reshape(B * TOPK),
                        axis=0)
    else:
        wide = _run_sc_gather(z.reshape(B * N, 2 * ZD),
                              gidx.reshape(B * TOPK))
    z_topk = wide[:, :ZD].reshape(B, TOPK, ZD)
    return (dlog, mu, sigma, z_topk, mu_topk, sigma_topk)


# speedup vs baseline: 5.7290x; 1.0174x over previous
import functools

import jax
import jax.numpy as jnp
from jax import lax
from jax.experimental import pallas as pl
from jax.experimental.pallas import tpu as pltpu
from jax.experimental.pallas import tpu_sc as plsc

B, N, XD, ZD, NC, SN, TOPK = 64, 8192, 64, 64, 4, 50, 256
P = 2 * NC
F = N * P
TN = 2048
CAND = 512
NCORES, NSUB = 2, 16
NW = NCORES * NSUB
BISECT_ITERS = 24

_f32 = jnp.float32
_i32 = jnp.int32


def _proxy_kernel(proxies_ref, eps_ref, wd_ref, bd_ref,
                  mu_ref, sigma_ref, zpn_ref, dlog_ref):
    pr = proxies_ref[...]
    mu = pr[:, :ZD]
    x = pr[:, ZD:]
    sigma = jnp.maximum(x, 0.0) + jnp.log1p(jnp.exp(-jnp.abs(x)))
    mu_ref[...] = mu
    sigma_ref[...] = sigma
    eps = eps_ref[...]
    zps = mu[:, None, :] + sigma[:, None, :] * eps
    zp = jnp.mean(zps, axis=1)
    nrm = jnp.sqrt(jnp.sum(zp * zp, axis=1, keepdims=True))
    zpn_ref[...] = zp / jnp.maximum(nrm, 1e-12)
    flat = zps.reshape(P * SN, ZD)
    logits = jnp.dot(flat, wd_ref[...], preferred_element_type=_f32)
    logits = logits + bd_ref[...]
    dlog_ref[...] = jnp.mean(logits.reshape(P, SN, NC), axis=1)


def _run_proxy(proxies, eps, Wd, bd):
    return pl.pallas_call(
        _proxy_kernel,
        out_shape=(
            jax.ShapeDtypeStruct((P, ZD), _f32),
            jax.ShapeDtypeStruct((P, ZD), _f32),
            jax.ShapeDtypeStruct((P, ZD), _f32),
            jax.ShapeDtypeStruct((P, NC), _f32),
        ),
    )(proxies, eps, Wd, bd.reshape(1, NC))


def _mlp_kernel(x_ref, w1_ref, b1_ref, w2_ref, b2_ref, w3_ref, b3_ref,
                zpn_ref, z_ref, att_ref):
    xt = x_ref[0]
    h = jnp.maximum(jnp.dot(xt, w1_ref[...], preferred_element_type=_f32)
                    + b1_ref[...], 0.0)
    h = jnp.maximum(jnp.dot(h, w2_ref[...], preferred_element_type=_f32)
                    + b2_ref[...], 0.0)
    z = jnp.dot(h, w3_ref[...], preferred_element_type=_f32) + b3_ref[...]
    z_ref[0] = jnp.concatenate([z, z], axis=1)
    nrm = jnp.sqrt(jnp.sum(z * z, axis=1, keepdims=True))
    zn = z / jnp.maximum(nrm, 1e-12)
    att = lax.dot_general(zn, zpn_ref[...], (((1,), (1,)), ((), ())),
                          preferred_element_type=_f32)
    att_ref[0] = att.T


def _run_mlp(x, W1, b1, W2, b2, W3, b3, zpn):
    grid = (B, N // TN)
    return pl.pallas_call(
        _mlp_kernel,
        grid=grid,
        in_specs=[
            pl.BlockSpec((1, TN, XD), lambda b, j: (b, j, 0)),
            pl.BlockSpec((XD, 2 * ZD), lambda b, j: (0, 0)),
            pl.BlockSpec((1, 2 * ZD), lambda b, j: (0, 0)),
            pl.BlockSpec((2 * ZD, 2 * ZD), lambda b, j: (0, 0)),
            pl.BlockSpec((1, 2 * ZD), lambda b, j: (0, 0)),
            pl.BlockSpec((2 * ZD, ZD), lambda b, j: (0, 0)),
            pl.BlockSpec((1, ZD), lambda b, j: (0, 0)),
            pl.BlockSpec((P, ZD), lambda b, j: (0, 0)),
        ],
        out_specs=[
            pl.BlockSpec((1, TN, 2 * ZD), lambda b, j: (b, j, 0)),
            pl.BlockSpec((1, P, TN), lambda b, j: (b, 0, j)),
        ],
        out_shape=(
            jax.ShapeDtypeStruct((B, N, 2 * ZD), _f32),
            jax.ShapeDtypeStruct((B, P, N), _f32),
        ),
        compiler_params=pltpu.CompilerParams(
            dimension_semantics=("parallel", "parallel")),
    )(x, W1, b1.reshape(1, -1), W2, b2.reshape(1, -1), W3, b3.reshape(1, -1),
      zpn)


def _bisect_kernel(att_ref, thr_ref):
    a = att_ref[...]
    rb = a.shape[0]
    k = jnp.float32(TOPK)

    def body(_, carry):
        lo, hi = carry
        mid = 0.5 * (lo + hi)
        cnt = jnp.sum((a > mid).astype(_f32), axis=1, keepdims=True)
        take = cnt >= k
        return jnp.where(take, mid, lo), jnp.where(take, hi, mid)

    lo0 = jnp.full((rb, 1), -1.01, _f32)
    hi0 = jnp.full((rb, 1), 1.01, _f32)
    lo, _ = lax.fori_loop(0, BISECT_ITERS, body, (lo0, hi0))
    thr_ref[...] = jnp.broadcast_to(lo, (rb, 16))


def _run_bisect(att_flat):
    rb = 8
    return pl.pallas_call(
        _bisect_kernel,
        grid=(B // rb,),
        in_specs=[pl.BlockSpec((rb, F), lambda i: (i, 0))],
        out_specs=pl.BlockSpec((rb, 16), lambda i: (i, 0)),
        out_shape=jax.ShapeDtypeStruct((B, 16), _f32),
        compiler_params=pltpu.CompilerParams(
            dimension_semantics=("arbitrary",)),
    )(att_flat)


def _sc_compact_body(att_hbm, thr_hbm, cv_hbm, ci_hbm,
                     buf_v, thr_v, cv_v, ci_v):
    wid = lax.axis_index("s") * NCORES + lax.axis_index("c")
    half = F // 2
    lane = lax.iota(_i32, 16)

    def one_batch(bi, _):
        b = wid * (B // NW) + bi
        pltpu.sync_copy(thr_hbm.at[b], thr_v)
        thrv = thr_v[...]

        def init_body(i, _c):
            cv_v[pl.ds(i * 16, 16)] = jnp.full((16,), -2.0, _f32)
            ci_v[pl.ds(i * 16, 16)] = jnp.zeros((16,), _i32)
            return 0

        lax.fori_loop(0, CAND // 16, init_body, 0)

        def scan_half(h, off):
            pltpu.sync_copy(att_hbm.at[b, pl.ds(h * half, half)], buf_v)

            def chunk(i, off):
                v = buf_v[pl.ds(i * 16, 16)]
                m = v > thrv

                def append(off):
                    mi = m.astype(_i32)
                    pos = jnp.minimum(off + plsc.cumsum(mi) - 1, CAND - 1)
                    fidx = h * half + i * 16 + lane
                    plsc.store_scatter(cv_v, [pos], v, mask=m)
                    plsc.store_scatter(ci_v, [pos], fidx, mask=m)
                    return off + jnp.sum(mi)

                return lax.cond(jnp.any(m), append, lambda off: off, off)

            return lax.fori_loop(0, half // 16, chunk, off)

        lax.fori_loop(0, 2, scan_half, 0)
        pltpu.sync_copy(cv_v, cv_hbm.at[b])
        pltpu.sync_copy(ci_v, ci_hbm.at[b])
        return 0

    lax.fori_loop(0, B // NW, one_batch, 0)


def _run_sc_compact(att_flat, thr):
    mesh = plsc.VectorSubcoreMesh(core_axis_name="c", subcore_axis_name="s",
                                  num_cores=NCORES, num_subcores=NSUB)
    return pl.kernel(
        _sc_compact_body,
        out_type=(
            jax.ShapeDtypeStruct((B, CAND), _f32),
            jax.ShapeDtypeStruct((B, CAND), _i32),
        ),
        mesh=mesh,
        scratch_types=[
            pltpu.VMEM((F // 2,), _f32),
            pltpu.VMEM((16,), _f32),
            pltpu.VMEM((CAND,), _f32),
            pltpu.VMEM((CAND,), _i32),
        ],
        compiler_params=pltpu.CompilerParams(needs_layout_passes=False),
    )(att_flat, thr)


def _rotl(x, s):
    return jnp.concatenate([x[:, s:], x[:, :s]], axis=1)


def _bitonic_desc_ref(vrf, frf):
    L = CAND
    ii = lax.broadcasted_iota(_i32, (1, L), 1)
    k = 2
    while k <= L:
        j = k // 2
        while j >= 1:
            v = vrf[...]
            f = frf[...]
            low = (ii & j) == 0
            fwd = (ii & k) == 0
            pv = jnp.where(low, _rotl(v, j), _rotl(v, L - j))
            pf = jnp.where(low, _rotl(f, j), _rotl(f, L - j))
            i_first = (v > pv) | ((v == pv) & (f < pf))
            take_mine = jnp.logical_xor(jnp.logical_xor(low, i_first), fwd)
            vrf[...] = jnp.where(take_mine, v, pv)
            frf[...] = jnp.where(take_mine, f, pf)
            j //= 2
        k *= 2


def _select_kernel(cv_ref, ci_ref, mu_ref, sg_ref,
                   gidx_ref, mut_ref, sgt_ref, vs_ref, fs_ref):
    v = cv_ref[...]
    fi = ci_ref[...]
    vs_ref[...] = v
    fs_ref[...] = fi
    _bitonic_desc_ref(vs_ref, fs_ref)
    f2 = fs_ref[...]
    n_sel = f2[:, :TOPK] & (N - 1)
    boff = lax.broadcasted_iota(_i32, (B, TOPK), 0) * N
    gidx_ref[...] = n_sel + boff
    vs_ref[...] = v
    fs_ref[...] = (fi & (N - 1)) * P + (fi >> 13)
    _bitonic_desc_ref(vs_ref, fs_ref)
    p_sel = fs_ref[...][:, :TOPK] & (P - 1)
    counts = [jnp.sum((p_sel == p).astype(_f32), axis=1, keepdims=True)
              for p in range(P)]
    best = counts[0]
    besti = jnp.zeros((B, 1), _i32)
    for p in range(1, P):
        m = counts[p] > best
        best = jnp.where(m, counts[p], best)
        besti = jnp.where(m, p, besti)
    mu = mu_ref[...]
    sg = sg_ref[...]
    mut = jnp.zeros((B, ZD), _f32)
    sgt = jnp.zeros((B, ZD), _f32)
    for p in range(P):
        m = besti == p
        mut = mut + jnp.where(m, mu[p:p + 1, :], 0.0)
        sgt = sgt + jnp.where(m, sg[p:p + 1, :], 0.0)
    mut_ref[...] = mut
    sgt_ref[...] = sgt


def _run_select(cv, ci, mu, sigma):
    return pl.pallas_call(
        _select_kernel,
        out_shape=(
            jax.ShapeDtypeStruct((B, TOPK), _i32),
            jax.ShapeDtypeStruct((B, ZD), _f32),
            jax.ShapeDtypeStruct((B, ZD), _f32),
        ),
        scratch_shapes=[
            pltpu.VMEM((B, CAND), _f32),
            pltpu.VMEM((B, CAND), _i32),
        ],
    )(cv, ci, mu, sigma)


def _sc_gather_body(z_hbm, idx_hbm, out_hbm, idx_v, rows_v, sem):
    wid = lax.axis_index("s") * NCORES + lax.axis_index("c")
    per = (B * TOPK) // NW
    base = wid * per
    pltpu.sync_copy(idx_hbm.at[pl.ds(base, per)], idx_v)
    pltpu.async_copy(z_hbm.at[idx_v], rows_v, sem).wait()
    pltpu.sync_copy(rows_v, out_hbm.at[pl.ds(base, per)])


def _run_sc_gather(z_rows, gidx_flat):
    per = (B * TOPK) // NW
    mesh = plsc.VectorSubcoreMesh(core_axis_name="c", subcore_axis_name="s",
                                  num_cores=NCORES, num_subcores=NSUB)
    return pl.kernel(
        _sc_gather_body,
        out_type=jax.ShapeDtypeStruct((B * TOPK, 2 * ZD), _f32),
        mesh=mesh,
        scratch_types=[
            pltpu.VMEM((per,), _i32),
            pltpu.VMEM((per, 2 * ZD), _f32),
            pltpu.SemaphoreType.DMA,
        ],
        compiler_params=pltpu.CompilerParams(needs_layout_passes=False),
    )(z_rows, gidx_flat)


def kernel(x, W1, b1, W2, b2, W3, b3, proxies, Wd, bd, eps):
    mu, sigma, zpn_unused, dlog = _run_proxy(proxies, eps, Wd, bd)
    mu_p = proxies[:, :ZD]
    sg_p = jax.nn.softplus(proxies[:, ZD:])
    zps = mu_p[:, None, :] + sg_p[:, None, :] * eps
    zp = jnp.mean(zps, axis=1)
    zpn = zp / jnp.maximum(jnp.linalg.norm(zp, axis=1, keepdims=True), 1e-12)
    z, att_t = _run_mlp(x, W1, b1, W2, b2, W3, b3, zpn)
    att_flat = att_t.reshape(B, F)
    thr = _run_bisect(att_flat)
    cv, ci = _run_sc_compact(att_flat, thr)
    gidx, mu_topk, sigma_topk = _run_select(cv, ci, mu, sigma)
    wide = _run_sc_gather(z.reshape(B * N, 2 * ZD), gidx.reshape(B * TOPK))
    z_topk = wide[:, :ZD].reshape(B, TOPK, ZD)
    return (dlog, mu, sigma, z_topk, mu_topk, sigma_topk)
